# Initial kernel scaffold; baseline (speedup 1.0000x reference)
import jax
import jax.numpy as jnp
from jax.experimental import pallas as pl


def kernel(x, edge_index, W1, a_src1, a_dst1, b1, W2, a_src2, a_dst2, b2):
    raise NotImplementedError("write your pallas kernel here")


# speedup vs baseline: 14.5729x; 14.5729x over previous
import functools

import jax
import jax.numpy as jnp
from jax import lax
from jax.experimental import pallas as pl
from jax.experimental.pallas import tpu as pltpu
from jax.experimental.pallas import tpu_sc as plsc

N = 10000
NP = 10240
E = 320000
NTILES = 16
CHUNK = 128
RING = 16
EP = 327680
TROWS = NP // NTILES
F = 128
R = 1000
GRID = N // R


def _mm1_body(x_ref, w_ref, asrc_ref, adst_ref, hs_ref, aso_ref, ado_ref):
    h = jnp.dot(x_ref[...], w_ref[...], preferred_element_type=jnp.float32)
    hs_ref[0] = h[:, :F]
    hs_ref[1] = h[:, F:]
    aso_ref[...] = jnp.sum(h * asrc_ref[...][None, :], axis=1, keepdims=True)
    ado_ref[...] = jnp.sum(h * adst_ref[...][None, :], axis=1, keepdims=True)


def _mm1(x, W, a_src, a_dst):
    return pl.pallas_call(
        _mm1_body,
        grid=(GRID,),
        in_specs=[
            pl.BlockSpec((R, 128), lambda i: (i, 0)),
            pl.BlockSpec((128, 256), lambda i: (0, 0)),
            pl.BlockSpec((256,), lambda i: (0,)),
            pl.BlockSpec((256,), lambda i: (0,)),
        ],
        out_specs=[
            pl.BlockSpec((2, R, F), lambda i: (0, i, 0)),
            pl.BlockSpec((R, 1), lambda i: (i, 0)),
            pl.BlockSpec((R, 1), lambda i: (i, 0)),
        ],
        out_shape=[
            jax.ShapeDtypeStruct((2, N, F), jnp.float32),
            jax.ShapeDtypeStruct((N, 1), jnp.float32),
            jax.ShapeDtypeStruct((N, 1), jnp.float32),
        ],
    )(x, W, a_src, a_dst)


def _mm2_body(h_ref, w_ref, asrc_ref, adst_ref, ho_ref, aso_ref, ado_ref):
    o = jnp.dot(h_ref[...], w_ref[...], preferred_element_type=jnp.float32)
    ho_ref[...] = o
    aso_ref[...] = jnp.sum(o * asrc_ref[...][None, :], axis=1, keepdims=True)
    ado_ref[...] = jnp.sum(o * adst_ref[...][None, :], axis=1, keepdims=True)


def _mm2(h, W, a_src, a_dst):
    return pl.pallas_call(
        _mm2_body,
        grid=(GRID,),
        in_specs=[
            pl.BlockSpec((R, 256), lambda i: (i, 0)),
            pl.BlockSpec((256, 128), lambda i: (0, 0)),
            pl.BlockSpec((128,), lambda i: (0,)),
            pl.BlockSpec((128,), lambda i: (0,)),
        ],
        out_specs=[
            pl.BlockSpec((R, 128), lambda i: (i, 0)),
            pl.BlockSpec((R, 1), lambda i: (i, 0)),
            pl.BlockSpec((R, 1), lambda i: (i, 0)),
        ],
        out_shape=[
            jax.ShapeDtypeStruct((N, 128), jnp.float32),
            jax.ShapeDtypeStruct((N, 1), jnp.float32),
            jax.ShapeDtypeStruct((N, 1), jnp.float32),
        ],
    )(h, W, a_src, a_dst)


def _fin_body(num_ref, den_ref, as_ref, ad_ref, b_ref, h_ref, out_ref, *,
              concat, last):
    a = as_ref[...] + ad_ref[...]
    wself = jnp.exp(jnp.where(a >= 0, a, 0.2 * a))
    den = den_ref[0] + den_ref[1] + wself + 1e-16
    if concat:
        num = jnp.concatenate([num_ref[0], num_ref[1]], axis=1)
        h = jnp.concatenate([h_ref[0], h_ref[1]], axis=1)
    else:
        num = num_ref[0] + num_ref[1]
        h = h_ref[...]
    o = (num + wself * h) / den + b_ref[...][None, :]
    if last:
        m = jnp.max(o, axis=1, keepdims=True)
        o = o - m
        out_ref[...] = o - jnp.log(jnp.sum(jnp.exp(o), axis=1, keepdims=True))
    else:
        out_ref[...] = jnp.maximum(o, 0.0)


def _fin(num, den, a_s, a_d, b, h, *, concat, last):
    FT = 2 * F if concat else F
    body = functools.partial(_fin_body, concat=concat, last=last)
    h_spec = (pl.BlockSpec((2, R, F), lambda i: (0, i, 0)) if concat
              else pl.BlockSpec((R, F), lambda i: (i, 0)))
    return pl.pallas_call(
        body,
        grid=(GRID,),
        in_specs=[
            pl.BlockSpec((2, R, F), lambda i: (0, i, 0)),
            pl.BlockSpec((2, R, 1), lambda i: (0, i, 0)),
            pl.BlockSpec((R, 1), lambda i: (i, 0)),
            pl.BlockSpec((R, 1), lambda i: (i, 0)),
            pl.BlockSpec((FT,), lambda i: (0,)),
            h_spec,
        ],
        out_specs=pl.BlockSpec((R, FT), lambda i: (i, 0)),
        out_shape=jax.ShapeDtypeStruct((N, FT), jnp.float32),
    )(num, den, a_s, a_d, b, h)


def _make_edge_pass(feature_split):
    nslices = NTILES if feature_split else 2 * NTILES
    chunks = EP // (nslices * CHUNK)
    groups = chunks // RING
    mesh = plsc.VectorSubcoreMesh(
        core_axis_name="c", subcore_axis_name="s", num_cores=2,
        num_subcores=16)

    def body(hs, as_h, ad_h, src_h, dst_h, num_o, den_o,
             as_v, ad_v, srcv, dstv, w_v, rows_v, zcol_v,
             num_sp, den_sp, gsem, ssem, dsem):
        c = lax.axis_index("c")
        s = lax.axis_index("s")
        tslice = s if feature_split else c * NTILES + s

        pltpu.sync_copy(as_h, as_v)
        pltpu.sync_copy(ad_h, ad_v)

        zero16 = jnp.zeros((16,), jnp.float32)

        def z_row(r, carry):
            for k in range(F // 16):
                rows_v[r, pl.ds(k * 16, 16)] = zero16
            return carry
        lax.fori_loop(0, CHUNK, z_row, 0)

        def z_col(i, carry):
            zcol_v[pl.ds(i * 16, 16)] = zero16
            return carry
        lax.fori_loop(0, TROWS // 16, z_col, 0)

        for t in range(TROWS // CHUNK):
            pltpu.sync_copy(rows_v,
                            num_sp.at[pl.ds(s * TROWS + t * CHUNK, CHUNK)])
        pltpu.sync_copy(zcol_v, den_sp.at[pl.ds(s * TROWS, TROWS)])

        plsc.subcore_barrier()

        iota16 = lax.broadcasted_iota(jnp.int32, (16,), 0)
        base = tslice * chunks * CHUNK
        hsrc = hs.at[c] if feature_split else hs

        def group_body(g, carry):
            pltpu.sync_copy(src_h.at[tslice].at[pl.ds(g * RING, RING)], srcv)
            pltpu.sync_copy(dst_h.at[tslice].at[pl.ds(g * RING, RING)], dstv)

            def chunk_body(j, carry2):
                pltpu.async_copy(hsrc.at[srcv.at[j]], rows_v, gsem).wait()
                for i in range(CHUNK // 16):
                    s16 = srcv[j, pl.ds(i * 16, 16)]
                    d16 = dstv[j, pl.ds(i * 16, 16)]
                    av = plsc.load_gather(as_v, [s16])
                    bv = plsc.load_gather(ad_v, [d16])
                    e = av + bv
                    e = jnp.where(e >= 0, e, jnp.float32(0.2) * e)
                    w = jnp.exp(e)
                    eid = base + (g * RING + j) * CHUNK + i * 16 + iota16
                    w = jnp.where(eid < E, w, jnp.float32(0.0))
                    w_v[pl.ds(i * 16, 16)] = w

                if feature_split:
                    cidx = g * RING + j
                    scatter_den = (cidx * 2 < chunks) == (c == 0)
                else:
                    scatter_den = True

                @pl.when(scatter_den)
                def _():
                    pltpu.async_copy(w_v, den_sp.at[dstv.at[j]], dsem,
                                     add=True).wait()

                def srow(r, carry3):
                    wb = plsc.load_gather(
                        w_v, [jnp.zeros((16,), jnp.int32) + r])
                    for k in range(F // 16):
                        rows_v[r, pl.ds(k * 16, 16)] = (
                            rows_v[r, pl.ds(k * 16, 16)] * wb)
                    return carry3
                lax.fori_loop(0, CHUNK, srow, 0)

                pltpu.async_copy(rows_v, num_sp.at[dstv.at[j]], ssem,
                                 add=True).wait()
                return carry2
            lax.fori_loop(0, RING, chunk_body, 0)
            return carry
        lax.fori_loop(0, groups, group_body, 0)

        plsc.subcore_barrier()

        for t in range(TROWS // CHUNK):
            r0 = s * TROWS + t * CHUNK
            pltpu.sync_copy(num_sp.at[pl.ds(r0, CHUNK)], rows_v)
            pltpu.sync_copy(rows_v, num_o.at[c].at[pl.ds(r0, CHUNK)])

        pltpu.sync_copy(den_sp.at[pl.ds(s * TROWS, TROWS)], zcol_v)
        pltpu.sync_copy(zcol_v, den_o.at[c].at[pl.ds(s * TROWS, TROWS)])

    return pl.kernel(
        body,
        out_type=(jax.ShapeDtypeStruct((2, NP, F), jnp.float32),
                  jax.ShapeDtypeStruct((2, NP), jnp.float32)),
        mesh=mesh,
        compiler_params=pltpu.CompilerParams(needs_layout_passes=False),
        scratch_types=[
            pltpu.VMEM((NP,), jnp.float32),
            pltpu.VMEM((NP,), jnp.float32),
            pltpu.VMEM((RING, CHUNK), jnp.int32),
            pltpu.VMEM((RING, CHUNK), jnp.int32),
            pltpu.VMEM((CHUNK,), jnp.float32),
            pltpu.VMEM((CHUNK, F), jnp.float32),
            pltpu.VMEM((TROWS,), jnp.float32),
            pltpu.VMEM_SHARED((NP, F), jnp.float32),
            pltpu.VMEM_SHARED((NP,), jnp.float32),
            pltpu.SemaphoreType.DMA,
            pltpu.SemaphoreType.DMA,
            pltpu.SemaphoreType.DMA,
        ],
    )


_edge_l1 = _make_edge_pass(True)
_edge_l2 = _make_edge_pass(False)


def kernel(x, edge_index, W1, a_src1, a_dst1, b1, W2, a_src2, a_dst2, b2):
    src = edge_index[0].astype(jnp.int32)
    dst = edge_index[1].astype(jnp.int32)
    pad = jnp.zeros((EP - E,), jnp.int32)
    srcp = jnp.concatenate([src, pad])
    dstp = jnp.concatenate([dst, pad])
    src16 = srcp.reshape(NTILES, EP // (NTILES * CHUNK), CHUNK)
    dst16 = dstp.reshape(NTILES, EP // (NTILES * CHUNK), CHUNK)
    src32 = srcp.reshape(2 * NTILES, EP // (2 * NTILES * CHUNK), CHUNK)
    dst32 = dstp.reshape(2 * NTILES, EP // (2 * NTILES * CHUNK), CHUNK)

    def padded(a):
        return jnp.pad(a.reshape(N), (0, NP - N))

    hs1, as1, ad1 = _mm1(x, W1, a_src1, a_dst1)
    num1, den1 = _edge_l1(hs1, padded(as1), padded(ad1), src16, dst16)
    h2 = _fin(num1, den1[:, :N].reshape(2, N, 1), as1, ad1, b1, hs1,
              concat=True, last=False)
    ho, as2, ad2 = _mm2(h2, W2, a_src2, a_dst2)
    num2, den2 = _edge_l2(ho, padded(as2), padded(ad2), src32, dst32)
    return _fin(num2, den2[:, :N].reshape(2, N, 1), as2, ad2, b2, ho,
                concat=False, last=True)
